# Initial kernel scaffold; baseline (speedup 1.0000x reference)
import jax
import jax.numpy as jnp
from jax.experimental import pallas as pl


def kernel(x, edge_index, edge_attr, i1W1, i1b1, i1W2, i1b2, c1Wl, c1bl, c1Wr, g1, be1, i2W1, i2b1, i2W2, i2b2, c2Wl, c2bl, c2Wr, g2, be2, i3W1, i3b1, i3W2, i3b2, c3Wl, c3bl, c3Wr, g3, be3):
    raise NotImplementedError("write your pallas kernel here")


# speedup vs baseline: 3.5823x; 3.5823x over previous
import functools

import jax
import jax.numpy as jnp
from jax import lax
from jax.experimental import pallas as pl
from jax.experimental.pallas import tpu as pltpu
from jax.experimental.pallas import tpu_sc as plsc

N = 10000
E = 320000
D = 128
NP_ = 10240

NC = 2
NS = 16
NW = NC * NS
EPW = E // NW
NPS = NP_ // NS

C_I, NCH_I = 80, 125
C_G, NCH_G = 125, 80

_MESH = plsc.VectorSubcoreMesh(core_axis_name="c", subcore_axis_name="s")


def _wid():
    return lax.axis_index("c") * NS + lax.axis_index("s")


def _sc_segsum_linear_body(width):
    def body(vals_hbm, dst_hbm, zeros_hbm, tok_hbm, out_hbm, idx_v, vals_v, acc):
        c = lax.axis_index("c")
        s = lax.axis_index("s")
        wid = c * NS + s
        pltpu.sync_copy(zeros_hbm, acc.at[pl.ds(s * NPS, NPS)])
        pltpu.sync_copy(dst_hbm.at[wid], idx_v)
        plsc.subcore_barrier()

        @pl.loop(0, NCH_I)
        def _(j):
            pltpu.sync_copy(vals_hbm.at[pl.ds(wid * EPW + j * C_I, C_I)], vals_v)
            pltpu.sync_copy(vals_v, acc.at[idx_v.at[j]], add=True)

        plsc.subcore_barrier()
        pltpu.sync_copy(acc.at[pl.ds(s * NPS, NPS)],
                        out_hbm.at[c, pl.ds(s * NPS, NPS)])

    return body


def _make_sc_segsum_linear(width):
    return pl.kernel(
        _sc_segsum_linear_body(width),
        out_type=jax.ShapeDtypeStruct((NC, NP_, width), jnp.float32),
        mesh=_MESH,
        scratch_types=[
            pltpu.VMEM((NCH_I, C_I), jnp.int32),
            pltpu.VMEM((C_I, width), jnp.float32),
            pltpu.VMEM_SHARED((NP_, width), jnp.float32),
        ],
    )


def _sc_gather_segsum_body(h_hbm, src_hbm, dst_hbm, zeros_hbm, out_hbm,
                           srci_v, dsti_v, vals_v, acc, sem):
    c = lax.axis_index("c")
    s = lax.axis_index("s")
    wid = c * NS + s
    pltpu.sync_copy(zeros_hbm, acc.at[pl.ds(s * NPS, NPS)])
    pltpu.sync_copy(src_hbm.at[wid], srci_v)
    pltpu.sync_copy(dst_hbm.at[wid], dsti_v)
    plsc.subcore_barrier()

    @pl.loop(0, NCH_G)
    def _(j):
        pltpu.async_copy(h_hbm.at[srci_v.at[j]], vals_v, sem).wait()
        pltpu.sync_copy(vals_v, acc.at[dsti_v.at[j]], add=True)

    plsc.subcore_barrier()
    pltpu.sync_copy(acc.at[pl.ds(s * NPS, NPS)],
                    out_hbm.at[c, pl.ds(s * NPS, NPS)])


_sc_gather_segsum = pl.kernel(
    _sc_gather_segsum_body,
    out_type=jax.ShapeDtypeStruct((NC, NP_, D), jnp.float32),
    mesh=_MESH,
    scratch_types=[
        pltpu.VMEM((NCH_G, C_G), jnp.int32),
        pltpu.VMEM((NCH_G, C_G), jnp.int32),
        pltpu.VMEM((C_G, D), jnp.float32),
        pltpu.VMEM_SHARED((NP_, D), jnp.float32),
        pltpu.SemaphoreType.DMA,
    ],
)

_sc_segsum_128 = _make_sc_segsum_linear(D)


_BE = 2000


def _bfr(v):
    return v.astype(jnp.bfloat16).astype(jnp.float32)


def _tc_prep_body(ea_ref, W1_ref, b1_ref, u1_ref, u2_ref, u3_ref, w_ref):
    ea = ea_ref[...]
    sg = jax.nn.sigmoid(ea[:, 2:3])
    eab = _bfr(ea)
    W1b = _bfr(W1_ref[...])
    a = b1_ref[...]
    a = (a + eab[:, 0:1] * W1b[0:1, :] + eab[:, 1:2] * W1b[1:2, :]
         + eab[:, 2:3] * W1b[2:3, :] + eab[:, 3:4] * W1b[3:4, :])
    a = _bfr(jnp.maximum(a, 0.0)) * sg
    u1_ref[...] = a[:, 0:128]
    u2_ref[...] = a[:, 128:256]
    u3_ref[...] = a[:, 256:384]
    w_ref[...] = jnp.concatenate(
        [jnp.ones((_BE, 1), jnp.float32), sg,
         jnp.zeros((_BE, 126), jnp.float32)], axis=1)


def _tc_prep(ea, W1all, b1all):
    return pl.pallas_call(
        _tc_prep_body,
        grid=(E // _BE,),
        in_specs=[
            pl.BlockSpec((_BE, 4), lambda i: (i, 0)),
            pl.BlockSpec((4, 384), lambda i: (0, 0)),
            pl.BlockSpec((1, 384), lambda i: (0, 0)),
        ],
        out_specs=[
            pl.BlockSpec((_BE, D), lambda i: (i, 0)),
            pl.BlockSpec((_BE, D), lambda i: (i, 0)),
            pl.BlockSpec((_BE, D), lambda i: (i, 0)),
            pl.BlockSpec((_BE, D), lambda i: (i, 0)),
        ],
        out_shape=[
            jax.ShapeDtypeStruct((E, D), jnp.float32),
            jax.ShapeDtypeStruct((E, D), jnp.float32),
            jax.ShapeDtypeStruct((E, D), jnp.float32),
            jax.ShapeDtypeStruct((E, D), jnp.float32),
        ],
    )(ea, W1all, b1all)


def _hdot(a, b):
    return jnp.dot(a, b, preferred_element_type=jnp.float32,
                   precision=lax.Precision.HIGHEST)


_R = 2000
_GR = N // _R


def _tc_inject_body(x_ref, t_ref, q_ref, W2_ref, b2_ref, h_ref):
    t = t_ref[0] + t_ref[1]
    scnt = q_ref[0, :, 1:2] + q_ref[1, :, 1:2]
    h_ref[...] = x_ref[...] + _hdot(t, _bfr(W2_ref[...])) + scnt * b2_ref[...]


def _tc_inject(x, t, q, W2, b2):
    return pl.pallas_call(
        _tc_inject_body,
        grid=(_GR,),
        in_specs=[
            pl.BlockSpec((_R, D), lambda i: (i, 0)),
            pl.BlockSpec((2, _R, D), lambda i: (0, i, 0)),
            pl.BlockSpec((2, _R, D), lambda i: (0, i, 0)),
            pl.BlockSpec((D, D), lambda i: (0, 0)),
            pl.BlockSpec((1, D), lambda i: (0, 0)),
        ],
        out_specs=pl.BlockSpec((_R, D), lambda i: (i, 0)),
        out_shape=jax.ShapeDtypeStruct((N, D), jnp.float32),
    )(x, t, q, W2, b2)


def _tc_sage_body(a_ref, h_ref, q_ref, Wl_ref, bl_ref, Wr_ref,
                  y_ref, s1_ref, s2_ref):
    cnt = q_ref[0, :, 0:1] + q_ref[1, :, 0:1]
    agg = (a_ref[0] + a_ref[1]) / jnp.maximum(cnt, 1.0)
    y = (jnp.dot(agg, Wl_ref[...], preferred_element_type=jnp.float32)
         + bl_ref[...]
         + jnp.dot(h_ref[...], Wr_ref[...], preferred_element_type=jnp.float32))
    y_ref[...] = y
    s1_ref[...] = jnp.broadcast_to(jnp.sum(y, axis=0, keepdims=True),
                                   (8, D)).reshape(1, 8, D)
    s2_ref[...] = jnp.broadcast_to(jnp.sum(y * y, axis=0, keepdims=True),
                                   (8, D)).reshape(1, 8, D)


def _tc_sage(a, h, q, Wl, bl, Wr):
    return pl.pallas_call(
        _tc_sage_body,
        grid=(_GR,),
        in_specs=[
            pl.BlockSpec((2, _R, D), lambda i: (0, i, 0)),
            pl.BlockSpec((_R, D), lambda i: (i, 0)),
            pl.BlockSpec((2, _R, D), lambda i: (0, i, 0)),
            pl.BlockSpec((D, D), lambda i: (0, 0)),
            pl.BlockSpec((1, D), lambda i: (0, 0)),
            pl.BlockSpec((D, D), lambda i: (0, 0)),
        ],
        out_specs=[
            pl.BlockSpec((_R, D), lambda i: (i, 0)),
            pl.BlockSpec((1, 8, D), lambda i: (i, 0, 0)),
            pl.BlockSpec((1, 8, D), lambda i: (i, 0, 0)),
        ],
        out_shape=[
            jax.ShapeDtypeStruct((N, D), jnp.float32),
            jax.ShapeDtypeStruct((_GR, 8, D), jnp.float32),
            jax.ShapeDtypeStruct((_GR, 8, D), jnp.float32),
        ],
    )(a, h, q, Wl, bl, Wr)


def _bn_relu(y_ref, s1_ref, s2_ref, g_ref, be_ref):
    mu = jnp.sum(s1_ref[...], axis=(0, 1)).reshape(1, D) * (1.0 / (8 * N))
    ey2 = jnp.sum(s2_ref[...], axis=(0, 1)).reshape(1, D) * (1.0 / (8 * N))
    var = ey2 - mu * mu
    z = (y_ref[...] - mu) * lax.rsqrt(var + 1e-5) * g_ref[...] + be_ref[...]
    return jnp.maximum(z, 0.0)


def _tc_bn_inject_body(y_ref, s1_ref, s2_ref, g_ref, be_ref,
                       t_ref, q_ref, W2n_ref, b2n_ref, out_ref):
    z = _bn_relu(y_ref, s1_ref, s2_ref, g_ref, be_ref)
    t = t_ref[0] + t_ref[1]
    scnt = q_ref[0, :, 1:2] + q_ref[1, :, 1:2]
    out_ref[...] = z + _hdot(t, _bfr(W2n_ref[...])) + scnt * b2n_ref[...]


def _tc_bn_inject(y, s1, s2, g, be, t, q, W2n, b2n):
    return pl.pallas_call(
        _tc_bn_inject_body,
        grid=(_GR,),
        in_specs=[
            pl.BlockSpec((_R, D), lambda i: (i, 0)),
            pl.BlockSpec((_GR, 8, D), lambda i: (0, 0, 0)),
            pl.BlockSpec((_GR, 8, D), lambda i: (0, 0, 0)),
            pl.BlockSpec((1, D), lambda i: (0, 0)),
            pl.BlockSpec((1, D), lambda i: (0, 0)),
            pl.BlockSpec((2, _R, D), lambda i: (0, i, 0)),
            pl.BlockSpec((2, _R, D), lambda i: (0, i, 0)),
            pl.BlockSpec((D, D), lambda i: (0, 0)),
            pl.BlockSpec((1, D), lambda i: (0, 0)),
        ],
        out_specs=pl.BlockSpec((_R, D), lambda i: (i, 0)),
        out_shape=jax.ShapeDtypeStruct((N, D), jnp.float32),
    )(y, s1, s2, g, be, t, q, W2n, b2n)


def _tc_bn_fin_body(y_ref, s1_ref, s2_ref, g_ref, be_ref, out_ref):
    out_ref[...] = _bn_relu(y_ref, s1_ref, s2_ref, g_ref, be_ref)


def _tc_bn_fin(y, s1, s2, g, be):
    return pl.pallas_call(
        _tc_bn_fin_body,
        grid=(_GR,),
        in_specs=[
            pl.BlockSpec((_R, D), lambda i: (i, 0)),
            pl.BlockSpec((_GR, 8, D), lambda i: (0, 0, 0)),
            pl.BlockSpec((_GR, 8, D), lambda i: (0, 0, 0)),
            pl.BlockSpec((1, D), lambda i: (0, 0)),
            pl.BlockSpec((1, D), lambda i: (0, 0)),
        ],
        out_specs=pl.BlockSpec((_R, D), lambda i: (i, 0)),
        out_shape=jax.ShapeDtypeStruct((N, D), jnp.float32),
    )(y, s1, s2, g, be)


def kernel(x, edge_index, edge_attr, i1W1, i1b1, i1W2, i1b2, c1Wl, c1bl, c1Wr,
           g1, be1, i2W1, i2b1, i2W2, i2b2, c2Wl, c2bl, c2Wr, g2, be2, i3W1,
           i3b1, i3W2, i3b2, c3Wl, c3bl, c3Wr, g3, be3):
    src = edge_index[0]
    dst = edge_index[1]
    dst3i = dst.reshape(NW, NCH_I, C_I)
    dst3g = dst.reshape(NW, NCH_G, C_G)
    src3g = src.reshape(NW, NCH_G, C_G)

    W1all = jnp.concatenate([i1W1, i2W1, i3W1], axis=1)
    b1all = jnp.concatenate([i1b1, i2b1, i3b1]).reshape(1, 384)

    u1, u2, u3, w = _tc_prep(edge_attr, W1all, b1all)

    zeros128 = jnp.zeros((NPS, D), jnp.float32)

    t1 = _sc_segsum_128(u1, dst3i, zeros128, w)
    q = _sc_segsum_128(w, dst3i, zeros128, t1)

    r2 = lambda v: v.reshape(1, -1)

    h1 = _tc_inject(x, t1, q, i1W2, r2(i1b2))
    a1 = _sc_gather_segsum(h1, src3g, dst3g, zeros128)
    y1, s11, s21 = _tc_sage(a1, h1, q, c1Wl, r2(c1bl), c1Wr)
    t2 = _sc_segsum_128(u2, dst3i, zeros128, t1)
    h2 = _tc_bn_inject(y1, s11, s21, r2(g1), r2(be1), t2, q, i2W2, r2(i2b2))
    a2 = _sc_gather_segsum(h2, src3g, dst3g, zeros128)
    y2, s12, s22 = _tc_sage(a2, h2, q, c2Wl, r2(c2bl), c2Wr)
    t3 = _sc_segsum_128(u3, dst3i, zeros128, t2)
    h3 = _tc_bn_inject(y2, s12, s22, r2(g2), r2(be2), t3, q, i3W2, r2(i3b2))
    a3 = _sc_gather_segsum(h3, src3g, dst3g, zeros128)
    y3, s13, s23 = _tc_sage(a3, h3, q, c3Wl, r2(c3bl), c3Wr)
    out = _tc_bn_fin(y3, s13, s23, r2(g3), r2(be3))
    out, *_ = lax.optimization_barrier((out, t1, t2, t3, q, a1, a2, a3))
    return out
